# baseline (speedup 1.0000x reference)
import functools

import jax
import jax.numpy as jnp
from jax import lax
from jax.experimental import pallas as pl
from jax.experimental.pallas import tpu as pltpu
from jax.experimental.pallas import tpu_sc as plsc

_PATCH = 32
_D = 256
_VOCAB = 2 * _PATCH * _PATCH
_B = 32
_N = 8192
_TOK = _B * _N
_REPL = 32
_NROW = _VOCAB + _REPL * _PATCH

_NC = 2
_NS = 16
_L = 16
_NW = _NC * _NS
_TPW = _TOK // _NW
_K = 64
_G = 2 * _K
_NCH = _TPW // _K
_NBUF = 2


def _prep_body(emb_ref, g_ref, b_ref, tab_ref):
    e = emb_ref[...]
    mu = jnp.mean(e, axis=-1, keepdims=True)
    var = jnp.mean(jnp.square(e - mu), axis=-1, keepdims=True)
    tab_ref[0:_VOCAB, :] = (e - mu) / jnp.sqrt(var + 1e-5) * g_ref[...] + b_ref[...]
    t = lax.broadcasted_iota(jnp.int32, (_PATCH, _D // 2), 0).astype(jnp.float32)
    k = lax.broadcasted_iota(jnp.int32, (_PATCH, _D // 2), 1).astype(jnp.float32)
    freqs = jnp.exp((-jnp.log(10000.0) / (_D // 2)) * k)
    args = t * freqs
    sin_block = jnp.concatenate([jnp.sin(args), jnp.cos(args)], axis=-1)
    for r in range(_REPL):
        tab_ref[pl.ds(_VOCAB + r * _PATCH, _PATCH), :] = sin_block


def _prep_tables(emb_table, ln_gamma, ln_beta):
    return pl.pallas_call(
        _prep_body,
        out_shape=jax.ShapeDtypeStruct((_NROW, _D), jnp.float32),
    )(emb_table, ln_gamma.reshape(1, _D), ln_beta.reshape(1, _D))


_QT = 2048


def _sc_body(raw_hbm, tab_hbm, out_hbm,
             raw_v, idx_v, rows_v, res_v,
             sem_g0, sem_g1, sem_o0, sem_o1):
    wid = lax.axis_index("s") * _NC + lax.axis_index("c")
    base = wid * _TPW
    sem_gs = (sem_g0, sem_g1)
    sem_os = (sem_o0, sem_o1)

    for q in range(_TPW // _QT):
        pltpu.sync_copy(
            raw_hbm.at[pl.ds((base + q * _QT) * 4, _QT * 4)], raw_v)

        def id_body(g, carry):
            lane4 = lax.iota(jnp.int32, _L) * 4 + g * (_L * 4)
            t = plsc.load_gather(raw_v, [lane4])
            x = plsc.load_gather(raw_v, [lane4 + 1])
            y = plsc.load_gather(raw_v, [lane4 + 2])
            p = plsc.load_gather(raw_v, [lane4 + 3])
            tok = lax.iota(jnp.int32, _L) + (g * _L + q * _QT)
            cpos = (tok >> 6) * _G + (tok & (_K - 1))
            plsc.store_scatter(idx_v, [cpos], (p + y) * _PATCH + x)
            plsc.store_scatter(idx_v, [cpos + _K],
                               (tok & (_REPL - 1)) * _PATCH + t + _VOCAB)
            return carry
        lax.fori_loop(0, _QT // _L, id_body, 0)

    def buf_slice(b):
        return rows_v.at[pl.ds(b * _G, _G)]

    def res_slice(b):
        return res_v.at[pl.ds(b * _K, _K)]

    def issue_gather(c, b):
        pltpu.async_copy(tab_hbm.at[idx_v.at[pl.ds(c * _G, _G)]],
                         buf_slice(b), sem_gs[b])

    def wait_gather(c, b):
        pltpu.make_async_copy(tab_hbm.at[idx_v.at[pl.ds(c * _G, _G)]],
                              buf_slice(b), sem_gs[b]).wait()

    def out_slice(c):
        return out_hbm.at[pl.ds(base + c * _K, _K)]

    for c0 in range(_NBUF):
        issue_gather(c0, c0)

    def outer(cg, carry):
        for b in range(_NBUF):
            c = cg * _NBUF + b
            wait_gather(c, b)

            @pl.when(c >= _NBUF)
            def _():
                pltpu.make_async_copy(res_slice(b), out_slice(c - _NBUF),
                                      sem_os[b]).wait()

            def add_body(tg, inner):
                for l in range(_L):
                    row = b * _G + tg * _L + l
                    rrow = b * _K + tg * _L + l
                    for j in range(_D // 32):
                        ab_ln = plsc.bitcast(
                            rows_v[row, pl.ds(j * _L, _L)], jnp.bfloat16)
                        ab_sin = plsc.bitcast(
                            rows_v[row + _K, pl.ds(j * _L, _L)], jnp.bfloat16)
                        a1, b1 = plsc.unpack(ab_ln, format=plsc.PackFormat.INTERLEAVED)
                        a2, b2 = plsc.unpack(ab_sin, format=plsc.PackFormat.INTERLEAVED)
                        res_v[rrow, pl.ds(j * 32, _L)] = a1 + a2
                        res_v[rrow, pl.ds(j * 32 + _L, _L)] = b1 + b2
                return inner
            lax.fori_loop(0, _K // _L, add_body, 0)

            pltpu.async_copy(res_slice(b), out_slice(c), sem_os[b])

            cn = c + _NBUF

            @pl.when(cn < _NCH)
            def _():
                issue_gather(cn, b)
        return carry
    lax.fori_loop(0, _NCH // _NBUF, outer, 0)

    for c in range(_NCH - _NBUF, _NCH):
        b = c % _NBUF
        pltpu.make_async_copy(res_slice(b), out_slice(c), sem_os[b]).wait()


@functools.cache
def _sc_gather():
    return functools.partial(
        pl.kernel,
        out_type=jax.ShapeDtypeStruct((_TOK, _D), jnp.float32),
        mesh=plsc.VectorSubcoreMesh(core_axis_name="c", subcore_axis_name="s",
                                    num_cores=_NC, num_subcores=_NS),
        compiler_params=pltpu.CompilerParams(needs_layout_passes=False),
        scratch_types=[
            pltpu.VMEM((_QT * 4,), jnp.int32),
            pltpu.VMEM((_TPW * 2,), jnp.int32),
            pltpu.VMEM((_NBUF * _G, _D // 2), jnp.int32),
            pltpu.VMEM((_NBUF * _K, _D), jnp.float32),
            pltpu.SemaphoreType.DMA,
            pltpu.SemaphoreType.DMA,
            pltpu.SemaphoreType.DMA,
            pltpu.SemaphoreType.DMA,
        ],
    )(_sc_body)


@jax.jit
def kernel(input, emb_table, ln_gamma, ln_beta):
    tab = _prep_tables(emb_table, ln_gamma, ln_beta)
    tab_bf = (tab.reshape(_NROW, _D // 32, 2, _L)
              .swapaxes(2, 3).reshape(_NROW, _D // 2, 2)
              .astype(jnp.bfloat16))
    tab_i32 = lax.bitcast_convert_type(tab_bf, jnp.int32)
    raw = input.reshape(_TOK * 4)
    out = _sc_gather()(raw, tab_i32)
    return out.reshape(_B, _N, _D)


# speedup vs baseline: 1.1410x; 1.1410x over previous
import functools

import jax
import jax.numpy as jnp
from jax import lax
from jax.experimental import pallas as pl
from jax.experimental.pallas import tpu as pltpu
from jax.experimental.pallas import tpu_sc as plsc

_PATCH = 32
_D = 256
_VOCAB = 2 * _PATCH * _PATCH
_B = 32
_N = 8192
_TOK = _B * _N
_REPL = 64
_NROW = _VOCAB + _REPL * _PATCH

_NC = 2
_NS = 16
_L = 16
_NW = _NC * _NS
_TPW = _TOK // _NW
_K = 64
_G = 2 * _K
_NCH = _TPW // _K
_NBUF = 2


def _prep_body(emb_ref, g_ref, b_ref, tab_ref):
    e = emb_ref[...]
    mu = jnp.mean(e, axis=-1, keepdims=True)
    var = jnp.mean(jnp.square(e - mu), axis=-1, keepdims=True)
    tab_ref[0:_VOCAB, :] = (e - mu) / jnp.sqrt(var + 1e-5) * g_ref[...] + b_ref[...]
    t = lax.broadcasted_iota(jnp.int32, (_PATCH, _D // 2), 0).astype(jnp.float32)
    k = lax.broadcasted_iota(jnp.int32, (_PATCH, _D // 2), 1).astype(jnp.float32)
    freqs = jnp.exp((-jnp.log(10000.0) / (_D // 2)) * k)
    args = t * freqs
    sin_block = jnp.concatenate([jnp.sin(args), jnp.cos(args)], axis=-1)
    for r in range(_REPL):
        tab_ref[pl.ds(_VOCAB + r * _PATCH, _PATCH), :] = sin_block


def _prep_tables(emb_table, ln_gamma, ln_beta):
    return pl.pallas_call(
        _prep_body,
        out_shape=jax.ShapeDtypeStruct((_NROW, _D), jnp.float32),
    )(emb_table, ln_gamma.reshape(1, _D), ln_beta.reshape(1, _D))


_QT = 2048


def _sc_body(raw_hbm, tab_hbm, out_hbm,
             raw_v, idx_v, rows_v, res_v,
             sem_g0, sem_g1, sem_o0, sem_o1):
    wid = lax.axis_index("s") * _NC + lax.axis_index("c")
    base = wid * _TPW
    sem_gs = (sem_g0, sem_g1)
    sem_os = (sem_o0, sem_o1)

    for q in range(_TPW // _QT):
        pltpu.sync_copy(
            raw_hbm.at[pl.ds((base + q * _QT) * 4, _QT * 4)], raw_v)

        def id_body(g, carry):
            lane4 = lax.iota(jnp.int32, _L) * 4 + g * (_L * 4)
            t = plsc.load_gather(raw_v, [lane4])
            x = plsc.load_gather(raw_v, [lane4 + 1])
            y = plsc.load_gather(raw_v, [lane4 + 2])
            p = plsc.load_gather(raw_v, [lane4 + 3])
            tok = lax.iota(jnp.int32, _L) + (g * _L + q * _QT)
            cpos = (tok >> 6) * _G + (tok & (_K - 1))
            plsc.store_scatter(idx_v, [cpos], (p + y) * _PATCH + x)
            plsc.store_scatter(idx_v, [cpos + _K],
                               (tok & (_REPL - 1)) * _PATCH + t + _VOCAB)
            return carry
        lax.fori_loop(0, _QT // _L, id_body, 0)

    def buf_slice(b):
        return rows_v.at[pl.ds(b * _G, _G)]

    def res_slice(b):
        return res_v.at[pl.ds(b * _K, _K)]

    def issue_gather(c, b):
        pltpu.async_copy(tab_hbm.at[idx_v.at[pl.ds(c * _G, _G)]],
                         buf_slice(b), sem_gs[b])

    def wait_gather(c, b):
        pltpu.make_async_copy(tab_hbm.at[idx_v.at[pl.ds(c * _G, _G)]],
                              buf_slice(b), sem_gs[b]).wait()

    def out_slice(c):
        return out_hbm.at[pl.ds(base + c * _K, _K)]

    for c0 in range(_NBUF):
        issue_gather(c0, c0)

    def outer(cg, carry):
        for b in range(_NBUF):
            c = cg * _NBUF + b
            wait_gather(c, b)

            @pl.when(c >= _NBUF)
            def _():
                pltpu.make_async_copy(res_slice(b), out_slice(c - _NBUF),
                                      sem_os[b]).wait()

            def add_body(tg, inner):
                for l in range(_L):
                    row = b * _G + tg * _L + l
                    rrow = b * _K + tg * _L + l
                    for j in range(_D // _L):
                        sl = pl.ds(j * _L, _L)
                        res_v[rrow, sl] = rows_v[row, sl] + rows_v[row + _K, sl]
                return inner
            lax.fori_loop(0, _K // _L, add_body, 0)

            pltpu.async_copy(res_slice(b), out_slice(c), sem_os[b])

            cn = c + _NBUF

            @pl.when(cn < _NCH)
            def _():
                issue_gather(cn, b)
        return carry
    lax.fori_loop(0, _NCH // _NBUF, outer, 0)

    for c in range(_NCH - _NBUF, _NCH):
        b = c % _NBUF
        pltpu.make_async_copy(res_slice(b), out_slice(c), sem_os[b]).wait()


@functools.cache
def _sc_gather():
    return functools.partial(
        pl.kernel,
        out_type=jax.ShapeDtypeStruct((_TOK, _D), jnp.float32),
        mesh=plsc.VectorSubcoreMesh(core_axis_name="c", subcore_axis_name="s",
                                    num_cores=_NC, num_subcores=_NS),
        compiler_params=pltpu.CompilerParams(needs_layout_passes=False),
        scratch_types=[
            pltpu.VMEM((_QT * 4,), jnp.int32),
            pltpu.VMEM((_TPW * 2,), jnp.int32),
            pltpu.VMEM((_NBUF * _G, _D), jnp.float32),
            pltpu.VMEM((_NBUF * _K, _D), jnp.float32),
            pltpu.SemaphoreType.DMA,
            pltpu.SemaphoreType.DMA,
            pltpu.SemaphoreType.DMA,
            pltpu.SemaphoreType.DMA,
        ],
    )(_sc_body)


@jax.jit
def kernel(input, emb_table, ln_gamma, ln_beta):
    tab = _prep_tables(emb_table, ln_gamma, ln_beta)
    raw = input.reshape(_TOK * 4)
    out = _sc_gather()(raw, tab)
    return out.reshape(_B, _N, _D)
